# Initial kernel scaffold; baseline (speedup 1.0000x reference)
import functools
import jax
import jax.numpy as jnp
from jax import lax
from jax.experimental import pallas as pl
from jax.experimental.pallas import tpu as pltpu
from jax.experimental.pallas import tpu_sc as plsc

R, C, KTOP = 128, 32768, 64
NC, NS, L = 2, 16, 16
NW = NC * NS
RPW = R // NW
NV = C // L
NB = 256


def _skey(v):
    b = plsc.bitcast(v, jnp.int32)
    m = lax.shift_right_arithmetic(b, 31)
    return b ^ (m & jnp.int32(0x7FFFFFFF))


def _body(x_hbm, o_hbm, row_v, cbuf, hist, sbuf):
    wid = lax.axis_index("s") * NC + lax.axis_index("c")
    lanes = lax.iota(jnp.int32, L)
    lane_base = lanes * NB
    ones = jnp.ones((L,), jnp.int32)
    zeros = jnp.zeros((L,), jnp.int32)

    def per_row(r, _):
        row = wid * RPW + r
        pltpu.sync_copy(x_hbm.at[pl.ds(row * C, C)], row_v)

        def z(i, _):
            hist[pl.ds(i * L, L)] = zeros
            return 0
        lax.fori_loop(0, (L * NB) // L, z, 0)

        def h(i, _):
            v = row_v[pl.ds(i * L, L)]
            d = lax.shift_right_arithmetic(_skey(v), 24) + 128
            plsc.addupdate_scatter(hist, [lane_base + d], ones)
            return 0
        lax.fori_loop(0, NV, h, 0)

        sbuf[pl.ds(NB, L)] = zeros
        carry = jnp.int32(0)
        s_vregs = [None] * (NB // L)
        for j in range((NB // L) - 1, -1, -1):
            acc = zeros
            for l in range(L):
                acc = acc + hist[pl.ds(l * NB + j * L, L)]
            s = lax.rev(jnp.cumsum(lax.rev(acc, (0,)), axis=0), (0,)) + carry
            sbuf[pl.ds(j * L, L)] = s
            s_vregs[j] = s
            carry = jnp.max(s)

        cge = jnp.int32(0)
        for j in range(NB // L):
            cge = cge + jnp.max(
                plsc.all_reduce_population_count(s_vregs[j] >= KTOP))
        d0 = cge - 1
        snext = plsc.load_gather(sbuf, [jnp.broadcast_to(d0 + 1, (L,))])
        krem = KTOP - jnp.max(snext)

        def cp(i, off):
            v = row_v[pl.ds(i * L, L)]
            sk = _skey(v)
            msk = (lax.shift_right_arithmetic(sk, 24) + 128) == d0
            plsc.store_compressed(cbuf.at[pl.ds(off, L)], sk, mask=msk)
            return off + jnp.max(plsc.all_reduce_population_count(msk))
        cnt = lax.fori_loop(0, NV, cp, jnp.int32(0))
        nv2 = (cnt + L - 1) // L

        prefix0 = lax.shift_left(d0 - 128, 24)

        def bit_step(t, prefix):
            cand = prefix | lax.shift_left(jnp.int32(1), 23 - t)

            def cstep(i, acc):
                sk = cbuf[pl.ds(i * L, L)]
                ge = (sk >= cand) & ((i * L + lanes) < cnt)
                return acc + jnp.max(plsc.all_reduce_population_count(ge))
            n = lax.fori_loop(0, nv2, cstep, jnp.int32(0))
            return jnp.where(n >= krem, cand, prefix)
        tkey = lax.fori_loop(0, 24, bit_step, prefix0)

        tb = jnp.where(tkey < 0, tkey ^ jnp.int32(0x7FFFFFFF), tkey)
        tf = plsc.bitcast(jnp.broadcast_to(tb, (L,)), jnp.float32)

        def mp(i, _):
            v = row_v[pl.ds(i * L, L)]
            row_v[pl.ds(i * L, L)] = jnp.where(v >= tf, v, jnp.float32(0))
            return 0
        lax.fori_loop(0, NV, mp, 0)

        pltpu.sync_copy(row_v, o_hbm.at[pl.ds(row * C, C)])
        return 0

    lax.fori_loop(0, RPW, per_row, 0)


@jax.jit
def kernel(x):
    mesh = plsc.VectorSubcoreMesh(core_axis_name="c", subcore_axis_name="s",
                                  num_cores=NC, num_subcores=NS)
    run = pl.kernel(
        _body,
        out_type=jax.ShapeDtypeStruct((R * C,), jnp.float32),
        mesh=mesh,
        scratch_types=[
            pltpu.VMEM((C,), jnp.float32),
            pltpu.VMEM((C + L,), jnp.int32),
            pltpu.VMEM((L * NB,), jnp.int32),
            pltpu.VMEM((NB + L,), jnp.int32),
        ],
    )
    return run(x.reshape(-1)).reshape(R, C)


# speedup vs baseline: 2.7051x; 2.7051x over previous
import functools
import jax
import jax.numpy as jnp
from jax import lax
from jax.experimental import pallas as pl
from jax.experimental.pallas import tpu as pltpu
from jax.experimental.pallas import tpu_sc as plsc

R, C, KTOP = 128, 32768, 64
NC, NS, L = 2, 16, 16
NW = NC * NS
RPW = R // NW
NV = C // L
NB = 256


def _skey(v):
    b = plsc.bitcast(v, jnp.int32)
    m = lax.shift_right_arithmetic(b, 31)
    return b ^ (m & jnp.int32(0x7FFFFFFF))


def _body(x_hbm, o_hbm, row_v, cbuf, hist, sbuf):
    wid = lax.axis_index("s") * NC + lax.axis_index("c")
    lanes = lax.iota(jnp.int32, L)
    lane_base = lanes * NB
    ones = jnp.ones((L,), jnp.int32)
    zeros = jnp.zeros((L,), jnp.int32)

    def per_row(r, _):
        row = wid * RPW + r
        pltpu.sync_copy(x_hbm.at[pl.ds(row * C, C)], row_v)

        def z(i, _):
            hist[pl.ds(i * L, L)] = zeros
            return 0
        lax.fori_loop(0, (L * NB) // L, z, 0)

        def h(i, _):
            v = row_v[pl.ds(i * L, L)]
            d = lax.shift_right_arithmetic(_skey(v), 24) + 128
            plsc.addupdate_scatter(hist, [lane_base + d], ones)
            return 0
        lax.fori_loop(0, NV, h, 0)

        sbuf[pl.ds(NB, L)] = zeros
        carry = jnp.int32(0)
        s_vregs = [None] * (NB // L)
        for j in range((NB // L) - 1, -1, -1):
            acc = zeros
            for l in range(L):
                acc = acc + hist[pl.ds(l * NB + j * L, L)]
            s = lax.rev(jnp.cumsum(lax.rev(acc, (0,)), axis=0), (0,)) + carry
            sbuf[pl.ds(j * L, L)] = s
            s_vregs[j] = s
            carry = jnp.max(s)

        cge = jnp.int32(0)
        for j in range(NB // L):
            cge = cge + jnp.max(
                plsc.all_reduce_population_count(s_vregs[j] >= KTOP))
        d0 = cge - 1
        snext = plsc.load_gather(sbuf, [jnp.broadcast_to(d0 + 1, (L,))])
        krem = KTOP - jnp.max(snext)

        def cp(i, off):
            v = row_v[pl.ds(i * L, L)]
            sk = _skey(v)
            msk = (lax.shift_right_arithmetic(sk, 24) + 128) == d0
            plsc.store_compressed(cbuf.at[pl.ds(off, L)], sk, mask=msk)
            return off + jnp.max(plsc.all_reduce_population_count(msk))
        cnt = lax.fori_loop(0, NV, cp, jnp.int32(0))
        nv2 = (cnt + L - 1) // L

        prefix0 = lax.shift_left(d0 - 128, 24)

        def bit_step(t, prefix):
            cand = prefix | lax.shift_left(jnp.int32(1), 23 - t)

            def cstep(i, acc):
                sk = cbuf[pl.ds(i * L, L)]
                ge = (sk >= cand) & ((i * L + lanes) < cnt)
                return acc + jnp.max(plsc.all_reduce_population_count(ge))
            n = lax.fori_loop(0, nv2, cstep, jnp.int32(0))
            return jnp.where(n >= krem, cand, prefix)
        tkey = lax.fori_loop(0, 24, bit_step, prefix0)

        tb = jnp.where(tkey < 0, tkey ^ jnp.int32(0x7FFFFFFF), tkey)
        tf = plsc.bitcast(jnp.broadcast_to(tb, (L,)), jnp.float32)

        def mp(i, _):
            v = row_v[pl.ds(i * L, L)]
            row_v[pl.ds(i * L, L)] = jnp.where(v >= tf, v, jnp.float32(0))
            return 0
        lax.fori_loop(0, NV, mp, 0)

        pltpu.sync_copy(row_v, o_hbm.at[pl.ds(row * C, C)])
        return 0

    lax.fori_loop(0, RPW, per_row, 0)


@jax.jit
def kernel(x):
    mesh = plsc.VectorSubcoreMesh(core_axis_name="c", subcore_axis_name="s",
                                  num_cores=NC, num_subcores=NS)
    run = pl.kernel(
        _body,
        out_type=jax.ShapeDtypeStruct((R * C,), jnp.float32),
        mesh=mesh,
        compiler_params=pltpu.CompilerParams(needs_layout_passes=False),
        scratch_types=[
            pltpu.VMEM((C,), jnp.float32),
            pltpu.VMEM((C + L,), jnp.int32),
            pltpu.VMEM((L * NB,), jnp.int32),
            pltpu.VMEM((NB + L,), jnp.int32),
        ],
    )
    return run(x.reshape(-1)).reshape(R, C)


# speedup vs baseline: 2.7843x; 1.0293x over previous
import functools
import jax
import jax.numpy as jnp
from jax import lax
from jax.experimental import pallas as pl
from jax.experimental.pallas import tpu as pltpu
from jax.experimental.pallas import tpu_sc as plsc

R, C, KTOP = 128, 32768, 64
NC, NS, L = 2, 16, 16
NW = NC * NS
RPW = R // NW
NV = C // L
NB = 256
CAP = 4096
UH = 8
UM = 8
UCP = 4
IMIN = -0x80000000


def _skey(v):
    b = plsc.bitcast(v, jnp.int32)
    m = lax.shift_right_arithmetic(b, 31)
    return b ^ (m & jnp.int32(0x7FFFFFFF))


def _body(x_hbm, o_hbm, rowa, rowb, rowc, cbuf, hist, sbuf,
          si0, si1, si2, si3, so0, so1, so2, so3):
    wid = lax.axis_index("s") * NC + lax.axis_index("c")
    base = wid * RPW
    lanes = lax.iota(jnp.int32, L)
    lane_base = lanes * NB
    ones = jnp.ones((L,), jnp.int32)
    zeros = jnp.zeros((L,), jnp.int32)

    bufs = [rowa, rowb, rowc, rowa]
    isems = [si0, si1, si2, si3]
    osems = [so0, so1, so2, so3]

    def z(i, _):
        for u in range(8):
            hist[pl.ds((i * 8 + u) * L, L)] = zeros
        return 0
    lax.fori_loop(0, (L * NB) // (8 * L), z, 0)

    in_h = [pltpu.async_copy(x_hbm.at[pl.ds((base + r) * C, C)],
                             bufs[r], isems[r]) for r in range(3)]
    out_h = [None] * RPW

    for r in range(RPW):
        buf = bufs[r]
        in_h[r].wait()

        def h(i, _):
            for u in range(UH):
                v = buf[pl.ds((i * UH + u) * L, L)]
                d = lax.shift_right_arithmetic(_skey(v), 24) + 128
                plsc.addupdate_scatter(hist, [lane_base + d], ones)
            return 0
        lax.fori_loop(0, NV // UH, h, 0)

        sbuf[pl.ds(NB, L)] = zeros

        def sj(t, carry_cge):
            carry, cge = carry_cge
            j = (NB // L) - 1 - t
            acc = zeros
            for l in range(L):
                acc = acc + hist[pl.ds(l * NB + j * L, L)]
                hist[pl.ds(l * NB + j * L, L)] = zeros
            s = lax.rev(jnp.cumsum(lax.rev(acc, (0,)), axis=0), (0,)) + carry
            sbuf[pl.ds(j * L, L)] = s
            cge = cge + jnp.max(plsc.all_reduce_population_count(s >= KTOP))
            return jnp.max(s), cge
        _, cge = lax.fori_loop(0, NB // L, sj, (jnp.int32(0), jnp.int32(0)))
        d0 = cge - 1
        snext = plsc.load_gather(sbuf, [jnp.broadcast_to(d0 + 1, (L,))])
        krem = KTOP - jnp.max(snext)

        def cp(i, off):
            for u in range(UCP):
                v = buf[pl.ds((i * UCP + u) * L, L)]
                sk = _skey(v)
                msk = (lax.shift_right_arithmetic(sk, 24) + 128) == d0
                pos = jnp.cumsum(msk.astype(jnp.int32))
                addr = jnp.minimum(off + pos - 1, CAP - 1)
                plsc.store_scatter(cbuf, [addr], sk, mask=msk)
                off = off + plsc.all_reduce_population_count(msk)
            return off
        offv = lax.fori_loop(0, NV // UCP, cp, zeros)
        cnt = jnp.max(offv)
        pbase = jnp.minimum(cnt, CAP)
        cbuf[pl.ds(pbase, L)] = jnp.broadcast_to(jnp.int32(IMIN), (L,))
        cbuf[pl.ds(pbase + L, L)] = jnp.broadcast_to(jnp.int32(IMIN), (L,))

        over = cnt > CAP
        nv_c = jnp.where(over, 0, (cnt + (2 * L - 1)) // (2 * L))
        nv_r = jnp.where(over, NV, 0)
        kq = jnp.where(over, KTOP, krem)
        prefix0 = lax.shift_left(d0 - 128, 24)

        def bit_step(t, prefix):
            cand = prefix | lax.shift_left(jnp.int32(1), 23 - t)

            def cstep(i, acc):
                s0 = cbuf[pl.ds(i * 2 * L, L)]
                s1 = cbuf[pl.ds((i * 2 + 1) * L, L)]
                acc = acc + plsc.all_reduce_population_count(s0 >= cand)
                return acc + plsc.all_reduce_population_count(s1 >= cand)
            nvec = lax.fori_loop(0, nv_c, cstep, zeros)

            def rstep(i, acc):
                sk = _skey(buf[pl.ds(i * L, L)])
                return acc + plsc.all_reduce_population_count(sk >= cand)
            nvec = lax.fori_loop(0, nv_r, rstep, nvec)
            return jnp.where(jnp.max(nvec) >= kq, cand, prefix)
        tkey = lax.fori_loop(0, 24, bit_step, prefix0)

        tb = jnp.where(tkey < 0, tkey ^ jnp.int32(0x7FFFFFFF), tkey)
        tf = plsc.bitcast(jnp.broadcast_to(tb, (L,)), jnp.float32)

        def mp(i, _):
            for u in range(UM):
                v = buf[pl.ds((i * UM + u) * L, L)]
                buf[pl.ds((i * UM + u) * L, L)] = \
                    jnp.where(v >= tf, v, jnp.float32(0))
            return 0
        lax.fori_loop(0, NV // UM, mp, 0)

        out_h[r] = pltpu.async_copy(buf, o_hbm.at[pl.ds((base + r) * C, C)],
                                    osems[r])
        if r == 1:
            out_h[0].wait()
            in_h.append(pltpu.async_copy(x_hbm.at[pl.ds((base + 3) * C, C)],
                                         bufs[3], isems[3]))

    for r in range(1, RPW):
        out_h[r].wait()


@jax.jit
def kernel(x):
    mesh = plsc.VectorSubcoreMesh(core_axis_name="c", subcore_axis_name="s",
                                  num_cores=NC, num_subcores=NS)
    run = pl.kernel(
        _body,
        out_type=jax.ShapeDtypeStruct((R * C,), jnp.float32),
        mesh=mesh,
        compiler_params=pltpu.CompilerParams(needs_layout_passes=False),
        scratch_types=[
            pltpu.VMEM((C,), jnp.float32),
            pltpu.VMEM((C,), jnp.float32),
            pltpu.VMEM((C,), jnp.float32),
            pltpu.VMEM((CAP + 2 * L,), jnp.int32),
            pltpu.VMEM((L * NB,), jnp.int32),
            pltpu.VMEM((NB + L,), jnp.int32),
            pltpu.SemaphoreType.DMA,
            pltpu.SemaphoreType.DMA,
            pltpu.SemaphoreType.DMA,
            pltpu.SemaphoreType.DMA,
            pltpu.SemaphoreType.DMA,
            pltpu.SemaphoreType.DMA,
            pltpu.SemaphoreType.DMA,
            pltpu.SemaphoreType.DMA,
        ],
    )
    return run(x.reshape(-1)).reshape(R, C)


# speedup vs baseline: 6.1197x; 2.1979x over previous
import functools
import jax
import jax.numpy as jnp
from jax import lax
from jax.experimental import pallas as pl
from jax.experimental.pallas import tpu as pltpu
from jax.experimental.pallas import tpu_sc as plsc

R, C, KTOP = 128, 32768, 64
NC, NS, L = 2, 16, 16
NW = NC * NS
RPW = R // NW
NV = C // L
NB = 256
CAP = 4096
UH = 8
UM = 8
UCP = 4
IMIN = -0x80000000


def _skey(v):
    b = plsc.bitcast(v, jnp.int32)
    m = lax.shift_right_arithmetic(b, 31)
    return b ^ (m & jnp.int32(0x7FFFFFFF))


def _body(x_hbm, o_hbm, rowa, rowb, rowc, cbuf, hist, sbuf,
          si0, si1, si2, si3, so0, so1, so2, so3):
    wid = lax.axis_index("s") * NC + lax.axis_index("c")
    base = wid * RPW
    lanes = lax.iota(jnp.int32, L)
    lane_base = lanes * NB
    ones = jnp.ones((L,), jnp.int32)
    zeros = jnp.zeros((L,), jnp.int32)

    bufs = [rowa, rowb, rowc, rowa]
    isems = [si0, si1, si2, si3]
    osems = [so0, so1, so2, so3]

    @plsc.parallel_loop(0, (L * NB) // L, unroll=8)
    def _(i):
        hist[pl.ds(i * L, L)] = zeros

    in_h = [pltpu.async_copy(x_hbm.at[pl.ds((base + r) * C, C)],
                             bufs[r], isems[r]) for r in range(3)]
    out_h = [None] * RPW

    for r in range(RPW):
        buf = bufs[r]
        in_h[r].wait()

        @plsc.parallel_loop(0, NV, unroll=UH)
        def _(i):
            v = buf[pl.ds(i * L, L)]
            d = lax.shift_right_arithmetic(_skey(v), 24) + 128
            plsc.addupdate_scatter(hist, [lane_base + d], ones)

        sbuf[pl.ds(NB, L)] = zeros

        def sj(t, carry_cge):
            carry, cge = carry_cge
            j = (NB // L) - 1 - t
            acc = zeros
            for l in range(L):
                acc = acc + hist[pl.ds(l * NB + j * L, L)]
                hist[pl.ds(l * NB + j * L, L)] = zeros
            s = lax.rev(jnp.cumsum(lax.rev(acc, (0,)), axis=0), (0,)) + carry
            sbuf[pl.ds(j * L, L)] = s
            cge = cge + jnp.max(plsc.all_reduce_population_count(s >= KTOP))
            return jnp.max(s), cge
        _, cge = lax.fori_loop(0, NB // L, sj, (jnp.int32(0), jnp.int32(0)))
        d0 = cge - 1
        snext = plsc.load_gather(sbuf, [jnp.broadcast_to(d0 + 1, (L,))])
        krem = KTOP - jnp.max(snext)

        @plsc.parallel_loop(0, NV, unroll=UCP, carry=zeros)
        def offv(i, off):
            v = buf[pl.ds(i * L, L)]
            sk = _skey(v)
            msk = (lax.shift_right_arithmetic(sk, 24) + 128) == d0
            pos = jnp.cumsum(msk.astype(jnp.int32))
            addr = jnp.minimum(off + pos - 1, CAP - 1)
            plsc.store_scatter(cbuf, [addr], sk, mask=msk)
            return off + plsc.all_reduce_population_count(msk)
        cnt = jnp.max(offv)
        pbase = jnp.minimum(cnt, CAP)
        cbuf[pl.ds(pbase, L)] = jnp.broadcast_to(jnp.int32(IMIN), (L,))
        cbuf[pl.ds(pbase + L, L)] = jnp.broadcast_to(jnp.int32(IMIN), (L,))

        over = cnt > CAP
        nv_c = jnp.where(over, 0, (cnt + (2 * L - 1)) // (2 * L))
        nv_r = jnp.where(over, NV, 0)
        kq = jnp.where(over, KTOP, krem)
        prefix0 = lax.shift_left(d0 - 128, 24)

        def bit_step(t, prefix):
            cand = prefix | lax.shift_left(jnp.int32(1), 23 - t)

            @plsc.parallel_loop(0, nv_c, carry=(zeros, zeros))
            def accs(i, acc):
                a0, a1 = acc
                s0 = cbuf[pl.ds(i * 2 * L, L)]
                s1 = cbuf[pl.ds((i * 2 + 1) * L, L)]
                return (a0 + plsc.all_reduce_population_count(s0 >= cand),
                        a1 + plsc.all_reduce_population_count(s1 >= cand))

            @plsc.parallel_loop(0, nv_r, unroll=4, carry=accs[0] + accs[1])
            def nvec(i, acc):
                sk = _skey(buf[pl.ds(i * L, L)])
                return acc + plsc.all_reduce_population_count(sk >= cand)
            return jnp.where(jnp.max(nvec) >= kq, cand, prefix)
        tkey = lax.fori_loop(0, 24, bit_step, prefix0)

        tb = jnp.where(tkey < 0, tkey ^ jnp.int32(0x7FFFFFFF), tkey)
        tf = plsc.bitcast(jnp.broadcast_to(tb, (L,)), jnp.float32)

        @plsc.parallel_loop(0, NV, unroll=UM)
        def _(i):
            v = buf[pl.ds(i * L, L)]
            buf[pl.ds(i * L, L)] = jnp.where(v >= tf, v, jnp.float32(0))

        out_h[r] = pltpu.async_copy(buf, o_hbm.at[pl.ds((base + r) * C, C)],
                                    osems[r])
        if r == 1:
            out_h[0].wait()
            in_h.append(pltpu.async_copy(x_hbm.at[pl.ds((base + 3) * C, C)],
                                         bufs[3], isems[3]))

    for r in range(1, RPW):
        out_h[r].wait()


@jax.jit
def kernel(x):
    mesh = plsc.VectorSubcoreMesh(core_axis_name="c", subcore_axis_name="s",
                                  num_cores=NC, num_subcores=NS)
    run = pl.kernel(
        _body,
        out_type=jax.ShapeDtypeStruct((R * C,), jnp.float32),
        mesh=mesh,
        compiler_params=pltpu.CompilerParams(needs_layout_passes=False),
        scratch_types=[
            pltpu.VMEM((C,), jnp.float32),
            pltpu.VMEM((C,), jnp.float32),
            pltpu.VMEM((C,), jnp.float32),
            pltpu.VMEM((CAP + 2 * L,), jnp.int32),
            pltpu.VMEM((L * NB,), jnp.int32),
            pltpu.VMEM((NB + L,), jnp.int32),
            pltpu.SemaphoreType.DMA,
            pltpu.SemaphoreType.DMA,
            pltpu.SemaphoreType.DMA,
            pltpu.SemaphoreType.DMA,
            pltpu.SemaphoreType.DMA,
            pltpu.SemaphoreType.DMA,
            pltpu.SemaphoreType.DMA,
            pltpu.SemaphoreType.DMA,
        ],
    )
    return run(x.reshape(-1)).reshape(R, C)


# speedup vs baseline: 8.3156x; 1.3588x over previous
import functools
import jax
import jax.numpy as jnp
from jax import lax
from jax.experimental import pallas as pl
from jax.experimental.pallas import tpu as pltpu
from jax.experimental.pallas import tpu_sc as plsc

R, C, KTOP = 128, 32768, 64
NC, NS, L = 2, 16, 16
NW = NC * NS
RPW = R // NW
NV = C // L
NB = 256
CAP = 4096
UH = 8
UM = 8
UCP = 4
IMIN = -0x80000000


def _skey(v):
    b = plsc.bitcast(v, jnp.int32)
    m = lax.shift_right_arithmetic(b, 31)
    return b ^ (m & jnp.int32(0x7FFFFFFF))


def _body(x_hbm, o_hbm, rowa, rowb, rowc, cbuf, hist, sbuf,
          si0, si1, si2, si3, so0, so1, so2, so3):
    wid = lax.axis_index("s") * NC + lax.axis_index("c")
    base = wid * RPW
    lanes = lax.iota(jnp.int32, L)
    lane_base = lanes * NB
    ones = jnp.ones((L,), jnp.int32)
    zeros = jnp.zeros((L,), jnp.int32)

    bufs = [rowa, rowb, rowc, rowa]
    isems = [si0, si1, si2, si3]
    osems = [so0, so1, so2, so3]

    @plsc.parallel_loop(0, (L * NB) // L, unroll=8)
    def _(i):
        hist[pl.ds(i * L, L)] = zeros

    in_h = [pltpu.async_copy(x_hbm.at[base + r], bufs[r], isems[r])
            for r in range(3)]
    out_h = [None] * RPW

    for r in range(RPW):
        buf = bufs[r]
        in_h[r].wait()

        @plsc.parallel_loop(0, NV, unroll=UH)
        def _(i):
            v = buf[pl.ds(i * L, L)]
            d = lax.shift_right_arithmetic(_skey(v), 24) + 128
            plsc.addupdate_scatter(hist, [lane_base + d], ones)

        sbuf[pl.ds(NB, L)] = zeros

        def sj(t, carry_cge):
            carry, cge = carry_cge
            j = (NB // L) - 1 - t
            acc = zeros
            for l in range(L):
                acc = acc + hist[pl.ds(l * NB + j * L, L)]
                hist[pl.ds(l * NB + j * L, L)] = zeros
            s = lax.rev(jnp.cumsum(lax.rev(acc, (0,)), axis=0), (0,)) + carry
            sbuf[pl.ds(j * L, L)] = s
            cge = cge + jnp.max(plsc.all_reduce_population_count(s >= KTOP))
            return jnp.max(s), cge
        _, cge = lax.fori_loop(0, NB // L, sj, (jnp.int32(0), jnp.int32(0)))
        d0 = cge - 1
        snext = plsc.load_gather(sbuf, [jnp.broadcast_to(d0 + 1, (L,))])
        krem = KTOP - jnp.max(snext)

        @plsc.parallel_loop(0, NV, unroll=UCP, carry=zeros)
        def offv(i, off):
            v = buf[pl.ds(i * L, L)]
            sk = _skey(v)
            msk = (lax.shift_right_arithmetic(sk, 24) + 128) == d0
            pos = jnp.cumsum(msk.astype(jnp.int32))
            addr = jnp.minimum(off + pos - 1, CAP - 1)
            plsc.store_scatter(cbuf, [addr], sk, mask=msk)
            return off + plsc.all_reduce_population_count(msk)
        cnt = jnp.max(offv)
        pbase = jnp.minimum(cnt, CAP)
        cbuf[pl.ds(pbase, L)] = jnp.broadcast_to(jnp.int32(IMIN), (L,))
        cbuf[pl.ds(pbase + L, L)] = jnp.broadcast_to(jnp.int32(IMIN), (L,))

        over = cnt > CAP
        nv_c = jnp.where(over, 0, (cnt + (2 * L - 1)) // (2 * L))
        nv_r = jnp.where(over, NV, 0)
        kq = jnp.where(over, KTOP, krem)
        prefix0 = lax.shift_left(d0 - 128, 24)

        def bit_step(t, prefix):
            cand = prefix | lax.shift_left(jnp.int32(1), 23 - t)

            @plsc.parallel_loop(0, nv_c, carry=(zeros, zeros))
            def accs(i, acc):
                a0, a1 = acc
                s0 = cbuf[pl.ds(i * 2 * L, L)]
                s1 = cbuf[pl.ds((i * 2 + 1) * L, L)]
                return (a0 + plsc.all_reduce_population_count(s0 >= cand),
                        a1 + plsc.all_reduce_population_count(s1 >= cand))

            @plsc.parallel_loop(0, nv_r, unroll=4, carry=accs[0] + accs[1])
            def nvec(i, acc):
                sk = _skey(buf[pl.ds(i * L, L)])
                return acc + plsc.all_reduce_population_count(sk >= cand)
            return jnp.where(jnp.max(nvec) >= kq, cand, prefix)
        tkey = lax.fori_loop(0, 24, bit_step, prefix0)

        tb = jnp.where(tkey < 0, tkey ^ jnp.int32(0x7FFFFFFF), tkey)
        tf = plsc.bitcast(jnp.broadcast_to(tb, (L,)), jnp.float32)

        @plsc.parallel_loop(0, NV, unroll=UM)
        def _(i):
            v = buf[pl.ds(i * L, L)]
            buf[pl.ds(i * L, L)] = jnp.where(v >= tf, v, jnp.float32(0))

        out_h[r] = pltpu.async_copy(buf, o_hbm.at[base + r], osems[r])
        if r == 1:
            out_h[0].wait()
            in_h.append(pltpu.async_copy(x_hbm.at[base + 3],
                                         bufs[3], isems[3]))

    for r in range(1, RPW):
        out_h[r].wait()


@jax.jit
def kernel(x):
    mesh = plsc.VectorSubcoreMesh(core_axis_name="c", subcore_axis_name="s",
                                  num_cores=NC, num_subcores=NS)
    run = pl.kernel(
        _body,
        out_type=jax.ShapeDtypeStruct((R, C), jnp.float32),
        mesh=mesh,
        compiler_params=pltpu.CompilerParams(needs_layout_passes=False),
        scratch_types=[
            pltpu.VMEM((C,), jnp.float32),
            pltpu.VMEM((C,), jnp.float32),
            pltpu.VMEM((C,), jnp.float32),
            pltpu.VMEM((CAP + 2 * L,), jnp.int32),
            pltpu.VMEM((L * NB,), jnp.int32),
            pltpu.VMEM((NB + L,), jnp.int32),
            pltpu.SemaphoreType.DMA,
            pltpu.SemaphoreType.DMA,
            pltpu.SemaphoreType.DMA,
            pltpu.SemaphoreType.DMA,
            pltpu.SemaphoreType.DMA,
            pltpu.SemaphoreType.DMA,
            pltpu.SemaphoreType.DMA,
            pltpu.SemaphoreType.DMA,
        ],
    )
    return run(x)


# speedup vs baseline: 8.8096x; 1.0594x over previous
import functools
import jax
import jax.numpy as jnp
from jax import lax
from jax.experimental import pallas as pl
from jax.experimental.pallas import tpu as pltpu
from jax.experimental.pallas import tpu_sc as plsc

R, C, KTOP = 128, 32768, 64
NC, NS, L = 2, 16, 16
NW = NC * NS
RPW = R // NW
NV = C // L
NB = 256
CAP = 4096
UH = 8
UM = 16
UCP = 4
IMIN = -0x80000000


def _skey(v):
    b = plsc.bitcast(v, jnp.int32)
    m = lax.shift_right_arithmetic(b, 31)
    return b ^ (m & jnp.int32(0x7FFFFFFF))


def _body(x_hbm, o_hbm, rowa, rowb, rowc, cbuf, hist, sbuf,
          si0, si1, si2, si3, so0, so1, so2, so3):
    wid = lax.axis_index("s") * NC + lax.axis_index("c")
    base = wid * RPW
    lanes = lax.iota(jnp.int32, L)
    lane_base = lanes * NB
    ones = jnp.ones((L,), jnp.int32)
    zeros = jnp.zeros((L,), jnp.int32)

    bufs = [rowa, rowb, rowc, rowa]
    isems = [si0, si1, si2, si3]
    osems = [so0, so1, so2, so3]

    @plsc.parallel_loop(0, (L * NB) // L, unroll=8)
    def _(i):
        hist[pl.ds(i * L, L)] = zeros

    in_h = [pltpu.async_copy(x_hbm.at[base + r], bufs[r], isems[r])
            for r in range(3)]
    out_h = [None] * RPW

    for r in range(RPW):
        buf = bufs[r]
        in_h[r].wait()

        @plsc.parallel_loop(0, NV, unroll=UH)
        def _(i):
            v = buf[pl.ds(i * L, L)]
            d = lax.shift_right_arithmetic(_skey(v), 24) + 128
            plsc.addupdate_scatter(hist, [lane_base + d], ones)

        sbuf[pl.ds(NB, L)] = zeros

        def sj(t, carry_cge):
            carry, cge = carry_cge
            j = (NB // L) - 1 - t
            acc = zeros
            for l in range(L):
                acc = acc + hist[pl.ds(l * NB + j * L, L)]
                hist[pl.ds(l * NB + j * L, L)] = zeros
            s = lax.rev(jnp.cumsum(lax.rev(acc, (0,)), axis=0), (0,)) + carry
            sbuf[pl.ds(j * L, L)] = s
            cge = cge + jnp.max(plsc.all_reduce_population_count(s >= KTOP))
            return jnp.max(s), cge
        _, cge = lax.fori_loop(0, NB // L, sj, (jnp.int32(0), jnp.int32(0)))
        d0 = cge - 1
        snext = plsc.load_gather(sbuf, [jnp.broadcast_to(d0 + 1, (L,))])
        krem = KTOP - jnp.max(snext)

        def _decode(kv):
            kb = jnp.broadcast_to(kv, (L,))
            kb = jnp.where(kb < 0, kb ^ jnp.int32(0x7FFFFFFF), kb)
            return plsc.bitcast(kb, jnp.float32)
        lo_f = _decode(lax.shift_left(d0 - 128, 24))
        hi_f = jnp.where(d0 == 255, jnp.float32(jnp.inf),
                         _decode(lax.shift_left(d0 - 127, 24)))

        @plsc.parallel_loop(0, NV, unroll=UCP, carry=zeros - 1)
        def offv(i, off):
            v = buf[pl.ds(i * L, L)]
            msk = (v >= lo_f) & (v < hi_f)
            addr = jnp.minimum(off + jnp.cumsum(msk.astype(jnp.int32)),
                               CAP - 1)
            plsc.store_scatter(cbuf, [addr], plsc.bitcast(v, jnp.int32),
                               mask=msk)
            return off + plsc.all_reduce_population_count(msk)
        cnt = jnp.max(offv) + 1
        pbase = jnp.minimum(cnt, CAP)
        cbuf[pl.ds(pbase, L)] = jnp.broadcast_to(jnp.int32(IMIN), (L,))
        cbuf[pl.ds(pbase + L, L)] = jnp.broadcast_to(jnp.int32(IMIN), (L,))

        over = (cnt > CAP) | (d0 <= 128)
        nv_c = jnp.where(over, 0, (cnt + (2 * L - 1)) // (2 * L))
        nv_r = jnp.where(over, NV, 0)
        kq = jnp.where(over, KTOP, krem)
        prefix0 = lax.shift_left(d0 - 128, 24)

        def bit_step(t, prefix):
            cand = prefix | lax.shift_left(jnp.int32(1), 23 - t)

            @plsc.parallel_loop(0, nv_c, carry=(zeros, zeros))
            def accs(i, acc):
                a0, a1 = acc
                s0 = cbuf[pl.ds(i * 2 * L, L)]
                s1 = cbuf[pl.ds((i * 2 + 1) * L, L)]
                return (a0 + plsc.all_reduce_population_count(s0 >= cand),
                        a1 + plsc.all_reduce_population_count(s1 >= cand))

            @plsc.parallel_loop(0, nv_r, unroll=4, carry=accs[0] + accs[1])
            def nvec(i, acc):
                sk = _skey(buf[pl.ds(i * L, L)])
                return acc + plsc.all_reduce_population_count(sk >= cand)
            return jnp.where(jnp.max(nvec) >= kq, cand, prefix)
        tkey = lax.fori_loop(0, 24, bit_step, prefix0)

        tb = jnp.where(tkey < 0, tkey ^ jnp.int32(0x7FFFFFFF), tkey)
        tf = plsc.bitcast(jnp.broadcast_to(tb, (L,)), jnp.float32)

        @plsc.parallel_loop(0, NV, unroll=UM)
        def _(i):
            v = buf[pl.ds(i * L, L)]
            buf[pl.ds(i * L, L)] = jnp.where(v >= tf, v, jnp.float32(0))

        out_h[r] = pltpu.async_copy(buf, o_hbm.at[base + r], osems[r])
        if r == 1:
            out_h[0].wait()
            in_h.append(pltpu.async_copy(x_hbm.at[base + 3],
                                         bufs[3], isems[3]))

    for r in range(1, RPW):
        out_h[r].wait()


@jax.jit
def kernel(x):
    mesh = plsc.VectorSubcoreMesh(core_axis_name="c", subcore_axis_name="s",
                                  num_cores=NC, num_subcores=NS)
    run = pl.kernel(
        _body,
        out_type=jax.ShapeDtypeStruct((R, C), jnp.float32),
        mesh=mesh,
        compiler_params=pltpu.CompilerParams(needs_layout_passes=False),
        scratch_types=[
            pltpu.VMEM((C,), jnp.float32),
            pltpu.VMEM((C,), jnp.float32),
            pltpu.VMEM((C,), jnp.float32),
            pltpu.VMEM((CAP + 2 * L,), jnp.int32),
            pltpu.VMEM((L * NB,), jnp.int32),
            pltpu.VMEM((NB + L,), jnp.int32),
            pltpu.SemaphoreType.DMA,
            pltpu.SemaphoreType.DMA,
            pltpu.SemaphoreType.DMA,
            pltpu.SemaphoreType.DMA,
            pltpu.SemaphoreType.DMA,
            pltpu.SemaphoreType.DMA,
            pltpu.SemaphoreType.DMA,
            pltpu.SemaphoreType.DMA,
        ],
    )
    return run(x)
